# initial kernel scaffold (unmeasured)
import functools

import jax
import jax.numpy as jnp
from jax import lax
from jax.experimental import pallas as pl
from jax.experimental.pallas import tpu as pltpu

N_DEV = 4


def kernel(x, w_mat):
    m_full, k_per = x.shape
    k_full, n = w_mat.shape
    m_per = m_full // N_DEV

    def body(x_ref, w_ref, out_ref, comm_ref, send_sems, recv_sems):
        my_pos = lax.axis_index("i")

        barrier_sem = pltpu.get_barrier_semaphore()
        for s in range(1, N_DEV):
            peer = (my_pos + s) % N_DEV
            pl.semaphore_signal(
                barrier_sem, inc=1,
                device_id=(peer,), device_id_type=pl.DeviceIdType.MESH,
            )
        pl.semaphore_wait(barrier_sem, N_DEV - 1)

        rdmas = []
        for s in range(1, N_DEV):
            dest = (my_pos + s) % N_DEV
            rdma = pltpu.make_async_remote_copy(
                src_ref=x_ref.at[pl.ds(dest * m_per, m_per), :],
                dst_ref=comm_ref.at[s - 1],
                send_sem=send_sems.at[s - 1],
                recv_sem=recv_sems.at[s - 1],
                device_id=(dest,),
                device_id_type=pl.DeviceIdType.MESH,
            )
            rdma.start()
            rdmas.append(rdma)

        x_own = x_ref[pl.ds(my_pos * m_per, m_per), :]
        w_own = w_ref[pl.ds(my_pos * k_per, k_per), :]
        out_ref[:, :] = jnp.dot(x_own, w_own, preferred_element_type=jnp.float32)

        for s in (1, 3, 2):
            rdmas[s - 1].wait_recv()
            src = (my_pos + (N_DEV - s)) % N_DEV
            w_blk = w_ref[pl.ds(src * k_per, k_per), :]
            out_ref[:, :] += jnp.dot(
                comm_ref[s - 1], w_blk, preferred_element_type=jnp.float32
            )

        out_ref[:, :] = jnp.maximum(out_ref[:, :], 0.0)

        for rdma in rdmas:
            rdma.wait_send()

        @functools.partial(pl.run_scoped, exit_sem=pltpu.SemaphoreType.REGULAR)
        def _(exit_sem):
            for s in range(1, N_DEV):
                peer = (my_pos + s) % N_DEV
                pl.semaphore_signal(
                    exit_sem, inc=1,
                    device_id=(peer,), device_id_type=pl.DeviceIdType.MESH,
                )
            pl.semaphore_wait(exit_sem, N_DEV - 1)

    return pl.pallas_call(
        body,
        out_shape=jax.ShapeDtypeStruct((m_per, n), jnp.float32),
        in_specs=[
            pl.BlockSpec(memory_space=pltpu.VMEM),
            pl.BlockSpec(memory_space=pltpu.VMEM),
        ],
        out_specs=pl.BlockSpec(memory_space=pltpu.VMEM),
        scratch_shapes=[
            pltpu.VMEM((N_DEV - 1, m_per, k_per), jnp.float32),
            pltpu.SemaphoreType.DMA((N_DEV - 1,)),
            pltpu.SemaphoreType.DMA((N_DEV - 1,)),
        ],
        compiler_params=pltpu.CompilerParams(collective_id=0),
    )(x, w_mat)


# baseline (device time: 128418 ns/iter reference)
import functools

import jax
import jax.numpy as jnp
from jax import lax
from jax.experimental import pallas as pl
from jax.experimental.pallas import tpu as pltpu

N_DEV = 4


def kernel(x, w_mat):
    m_full, k_per = x.shape
    k_full, n = w_mat.shape
    m_per = m_full // N_DEV

    def body(x_ref, w_ref, out_ref, comm_ref, own_ref, send_sems, recv_sems, copy_sem):
        my_pos = lax.axis_index("i")

        barrier_sem = pltpu.get_barrier_semaphore()
        for s in range(1, N_DEV):
            peer = (my_pos + s) % N_DEV
            pl.semaphore_signal(
                barrier_sem, inc=1,
                device_id=(peer,), device_id_type=pl.DeviceIdType.MESH,
            )
        pl.semaphore_wait(barrier_sem, N_DEV - 1)

        rdmas = []
        for s in range(1, N_DEV):
            dest = (my_pos + s) % N_DEV
            rdma = pltpu.make_async_remote_copy(
                src_ref=x_ref.at[pl.ds(dest * m_per, m_per), :],
                dst_ref=comm_ref.at[s - 1],
                send_sem=send_sems.at[s - 1],
                recv_sem=recv_sems.at[s - 1],
                device_id=(dest,),
                device_id_type=pl.DeviceIdType.MESH,
            )
            rdma.start()
            rdmas.append(rdma)

        own_copy = pltpu.make_async_copy(
            x_ref.at[pl.ds(my_pos * m_per, m_per), :], own_ref, copy_sem
        )
        own_copy.start()
        own_copy.wait()
        w_own = w_ref[pl.ds(my_pos * k_per, k_per), :]
        out_ref[:, :] = jnp.dot(
            own_ref[:, :], w_own, preferred_element_type=jnp.float32
        )

        for s in (1, 3, 2):
            rdmas[s - 1].wait_recv()
            src = (my_pos + (N_DEV - s)) % N_DEV
            w_blk = w_ref[pl.ds(src * k_per, k_per), :]
            out_ref[:, :] += jnp.dot(
                comm_ref[s - 1], w_blk, preferred_element_type=jnp.float32
            )

        out_ref[:, :] = jnp.maximum(out_ref[:, :], 0.0)

        for rdma in rdmas:
            rdma.wait_send()

        @functools.partial(pl.run_scoped, exit_sem=pltpu.SemaphoreType.REGULAR)
        def _(exit_sem):
            for s in range(1, N_DEV):
                peer = (my_pos + s) % N_DEV
                pl.semaphore_signal(
                    exit_sem, inc=1,
                    device_id=(peer,), device_id_type=pl.DeviceIdType.MESH,
                )
            pl.semaphore_wait(exit_sem, N_DEV - 1)

    return pl.pallas_call(
        body,
        out_shape=jax.ShapeDtypeStruct((m_per, n), jnp.float32),
        in_specs=[
            pl.BlockSpec(memory_space=pltpu.HBM),
            pl.BlockSpec(memory_space=pltpu.VMEM),
        ],
        out_specs=pl.BlockSpec(memory_space=pltpu.VMEM),
        scratch_shapes=[
            pltpu.VMEM((N_DEV - 1, m_per, k_per), jnp.float32),
            pltpu.VMEM((m_per, k_per), jnp.float32),
            pltpu.SemaphoreType.DMA((N_DEV - 1,)),
            pltpu.SemaphoreType.DMA((N_DEV - 1,)),
            pltpu.SemaphoreType.DMA,
        ],
        compiler_params=pltpu.CompilerParams(
            collective_id=0, vmem_limit_bytes=100 * 1024 * 1024
        ),
    )(x, w_mat)


# device time: 78585 ns/iter; 1.6341x vs baseline; 1.6341x over previous
import functools

import jax
import jax.numpy as jnp
from jax import lax
from jax.experimental import pallas as pl
from jax.experimental.pallas import tpu as pltpu

N_DEV = 4
SEND_ORDER = (1, 3, 2)
N_CHUNK = 4


def kernel(x, w_mat):
    m_full, k_per = x.shape
    k_full, n = w_mat.shape
    m_per = m_full // N_DEV

    def body(x_ref, w_ref, out_ref, comm_ref, stage_ref, xblk_ref,
             send_sems, recv_sems, copy_sem, own_sem):
        my_pos = lax.axis_index("i")

        barrier_sem = pltpu.get_barrier_semaphore()
        for s in range(1, N_DEV):
            peer = (my_pos + s) % N_DEV
            pl.semaphore_signal(
                barrier_sem, inc=1,
                device_id=(peer,), device_id_type=pl.DeviceIdType.MESH,
            )
        pl.semaphore_wait(barrier_sem, N_DEV - 1)

        rdmas = {}
        m_chunk = m_per // N_CHUNK
        for s in SEND_ORDER:
            dest = (my_pos + s) % N_DEV
            blk = pltpu.make_async_copy(
                x_ref.at[pl.ds(dest * m_per, m_per), :], xblk_ref, copy_sem
            )
            blk.start()
            blk.wait()
            stage_ref[s - 1, :, :] = xblk_ref[:, :].astype(jnp.bfloat16)
            for c in range(N_CHUNK):
                idx = (s - 1) * N_CHUNK + c
                rows = pl.ds(c * m_chunk, m_chunk)
                rdma = pltpu.make_async_remote_copy(
                    src_ref=stage_ref.at[s - 1, rows, :],
                    dst_ref=comm_ref.at[s - 1, rows, :],
                    send_sem=send_sems.at[idx],
                    recv_sem=recv_sems.at[idx],
                    device_id=(dest,),
                    device_id_type=pl.DeviceIdType.MESH,
                )
                rdma.start()
                rdmas[s, c] = rdma

        own = pltpu.make_async_copy(
            x_ref.at[pl.ds(my_pos * m_per, m_per), :], xblk_ref, own_sem
        )
        own.start()
        own.wait()
        w_own = w_ref[pl.ds(my_pos * k_per, k_per), :]
        out_ref[:, :] = jnp.dot(
            xblk_ref[:, :], w_own, preferred_element_type=jnp.float32
        )

        for c in range(N_CHUNK):
            rows = pl.ds(c * m_chunk, m_chunk)
            for s in SEND_ORDER:
                rdmas[s, c].wait_recv()
                src = (my_pos + (N_DEV - s)) % N_DEV
                w_blk = w_ref[pl.ds(src * k_per, k_per), :]
                contrib = jnp.dot(
                    comm_ref[s - 1, rows, :].astype(jnp.float32), w_blk,
                    preferred_element_type=jnp.float32,
                )
                if s == SEND_ORDER[-1]:
                    out_ref[rows, :] = jnp.maximum(out_ref[rows, :] + contrib, 0.0)
                else:
                    out_ref[rows, :] += contrib

        for s in SEND_ORDER:
            for c in range(N_CHUNK):
                rdmas[s, c].wait_send()

        @functools.partial(pl.run_scoped, exit_sem=pltpu.SemaphoreType.REGULAR)
        def _(exit_sem):
            for s in range(1, N_DEV):
                peer = (my_pos + s) % N_DEV
                pl.semaphore_signal(
                    exit_sem, inc=1,
                    device_id=(peer,), device_id_type=pl.DeviceIdType.MESH,
                )
            pl.semaphore_wait(exit_sem, N_DEV - 1)

    return pl.pallas_call(
        body,
        out_shape=jax.ShapeDtypeStruct((m_per, n), jnp.float32),
        in_specs=[
            pl.BlockSpec(memory_space=pltpu.HBM),
            pl.BlockSpec(memory_space=pltpu.VMEM),
        ],
        out_specs=pl.BlockSpec(memory_space=pltpu.VMEM),
        scratch_shapes=[
            pltpu.VMEM((N_DEV - 1, m_per, k_per), jnp.bfloat16),
            pltpu.VMEM((N_DEV - 1, m_per, k_per), jnp.bfloat16),
            pltpu.VMEM((m_per, k_per), jnp.float32),
            pltpu.SemaphoreType.DMA(((N_DEV - 1) * N_CHUNK,)),
            pltpu.SemaphoreType.DMA(((N_DEV - 1) * N_CHUNK,)),
            pltpu.SemaphoreType.DMA,
            pltpu.SemaphoreType.DMA,
        ],
        compiler_params=pltpu.CompilerParams(
            collective_id=0, vmem_limit_bytes=100 * 1024 * 1024
        ),
    )(x, w_mat)


# device time: 74100 ns/iter; 1.7330x vs baseline; 1.0605x over previous
import jax
import jax.numpy as jnp
from jax import lax
from jax.experimental import pallas as pl
from jax.experimental.pallas import tpu as pltpu

N_DEV = 4
SEND_ORDER = (1, 3, 2)
N_CHUNK = 8


def kernel(x, w_mat):
    m_full, k_per = x.shape
    k_full, n = w_mat.shape
    m_per = m_full // N_DEV

    def body(x_ref, w_ref, out_ref, comm_ref, stage_ref, xblk_ref, cbuf_ref,
             send_sems, recv_sems, copy_sems, own_sem):
        my_pos = lax.axis_index("i")

        barrier_sem = pltpu.get_barrier_semaphore()
        for s in range(1, N_DEV):
            peer = (my_pos + s) % N_DEV
            pl.semaphore_signal(
                barrier_sem, inc=1,
                device_id=(peer,), device_id_type=pl.DeviceIdType.MESH,
            )
        pl.semaphore_wait(barrier_sem, N_DEV - 1)

        own = pltpu.make_async_copy(
            x_ref.at[pl.ds(my_pos * m_per, m_per), :], xblk_ref, own_sem
        )
        own.start()

        m_chunk = m_per // N_CHUNK
        items = [(s, c) for c in range(N_CHUNK) for s in SEND_ORDER]
        rdmas = {}

        def chunk_copy(i):
            s, c = items[i]
            dest = (my_pos + s) % N_DEV
            return pltpu.make_async_copy(
                x_ref.at[pl.ds(dest * m_per + c * m_chunk, m_chunk), :],
                cbuf_ref.at[i % 2],
                copy_sems.at[i % 2],
            )

        chunk_copy(0).start()
        for i, (s, c) in enumerate(items):
            if i + 1 < len(items):
                chunk_copy(i + 1).start()
            chunk_copy(i).wait()
            rows = pl.ds(c * m_chunk, m_chunk)
            stage_ref[s - 1, rows, :] = cbuf_ref[i % 2].astype(jnp.bfloat16)
            dest = (my_pos + s) % N_DEV
            idx = (s - 1) * N_CHUNK + c
            rdma = pltpu.make_async_remote_copy(
                src_ref=stage_ref.at[s - 1, rows, :],
                dst_ref=comm_ref.at[s - 1, rows, :],
                send_sem=send_sems.at[idx],
                recv_sem=recv_sems.at[idx],
                device_id=(dest,),
                device_id_type=pl.DeviceIdType.MESH,
            )
            rdma.start()
            rdmas[s, c] = rdma

        own.wait()
        w_own = w_ref[pl.ds(my_pos * k_per, k_per), :]
        out_ref[:, :] = jnp.dot(
            xblk_ref[:, :], w_own, preferred_element_type=jnp.float32
        )

        for c in range(N_CHUNK):
            rows = pl.ds(c * m_chunk, m_chunk)
            for s in SEND_ORDER:
                rdmas[s, c].wait_recv()
                src = (my_pos + (N_DEV - s)) % N_DEV
                w_blk = w_ref[pl.ds(src * k_per, k_per), :]
                contrib = jnp.dot(
                    comm_ref[s - 1, rows, :].astype(jnp.float32), w_blk,
                    preferred_element_type=jnp.float32,
                )
                if s == SEND_ORDER[-1]:
                    out_ref[rows, :] = jnp.maximum(out_ref[rows, :] + contrib, 0.0)
                else:
                    out_ref[rows, :] += contrib

        for s in SEND_ORDER:
            for c in range(N_CHUNK):
                rdmas[s, c].wait_send()


    return pl.pallas_call(
        body,
        out_shape=jax.ShapeDtypeStruct((m_per, n), jnp.float32),
        in_specs=[
            pl.BlockSpec(memory_space=pltpu.HBM),
            pl.BlockSpec(memory_space=pltpu.VMEM),
        ],
        out_specs=pl.BlockSpec(memory_space=pltpu.VMEM),
        scratch_shapes=[
            pltpu.VMEM((N_DEV - 1, m_per, k_per), jnp.bfloat16),
            pltpu.VMEM((N_DEV - 1, m_per, k_per), jnp.bfloat16),
            pltpu.VMEM((m_per, k_per), jnp.float32),
            pltpu.VMEM((2, m_per // N_CHUNK, k_per), jnp.float32),
            pltpu.SemaphoreType.DMA(((N_DEV - 1) * N_CHUNK,)),
            pltpu.SemaphoreType.DMA(((N_DEV - 1) * N_CHUNK,)),
            pltpu.SemaphoreType.DMA((2,)),
            pltpu.SemaphoreType.DMA,
        ],
        compiler_params=pltpu.CompilerParams(
            collective_id=0, vmem_limit_bytes=100 * 1024 * 1024
        ),
    )(x, w_mat)
